# baseline (speedup 1.0000x reference)
import functools

import jax
import jax.numpy as jnp
from jax import lax
from jax.experimental import pallas as pl
from jax.experimental.pallas import tpu as pltpu
from jax.experimental.pallas import tpu_sc as plsc

B = 16384
V = 1000000
RN = 1000
D = 64
NC = 2
NS = 16
NW = NC * NS
BW = B // NW
IC = 128
NCH = BW // IC
CS = BW * 16
NT = CS // IC
CI = 16384


def _tc_tr_body(x_ref, o_ref):
    eye = jnp.eye(D, dtype=jnp.float32)
    y = lax.dot_general(x_ref[...], eye, (((0,), (0,)), ((), ())),
                        preferred_element_type=jnp.float32)
    yi = lax.bitcast_convert_type(y, jnp.int32)
    lo = lax.shift_right_logical(yi[:, 0:D // 2], 16)
    hi = yi[:, D // 2:D] & jnp.int32(-65536)
    w = hi | lo
    q = CI // 4
    for t in range(4):
        o_ref[:, t * 32:(t + 1) * 32] = w[t * q:(t + 1) * q, :]


def _tc_transpose(xt, n_rows):
    grid = (n_rows + CI - 1) // CI
    return pl.pallas_call(
        _tc_tr_body,
        grid=(grid,),
        in_specs=[pl.BlockSpec((D, CI), lambda b: (0, b))],
        out_specs=pl.BlockSpec((CI // 4, 128), lambda b: (b, 0)),
        out_shape=jax.ShapeDtypeStruct((grid * CI // 4, 128), jnp.int32),
    )(xt)


def _sc_body(h_idx_hbm, t_idx_hbm, r_idx_hbm, ent_hbm, rel_hbm, perm_hbm,
             out_hbm, hi_v, ti_v, ri_v, h_v, t_v, r_v, csum_v,
             perm_v, out_v, slab, sem):
    cid = lax.axis_index("c")
    sid = lax.axis_index("s")
    wid = sid * NC + cid
    row0 = wid * NCH

    pltpu.sync_copy(h_idx_hbm.at[pl.ds(row0, NCH)], hi_v)
    pltpu.sync_copy(t_idx_hbm.at[pl.ds(row0, NCH)], ti_v)
    pltpu.sync_copy(r_idx_hbm.at[pl.ds(row0, NCH)], ri_v)
    pltpu.sync_copy(perm_hbm, perm_v)

    copies = []
    for k in range(NCH):
        copies.append(pltpu.async_copy(
            ent_hbm.at[hi_v.at[k]], h_v.at[pl.ds(k * IC, IC)], sem))
        copies.append(pltpu.async_copy(
            ent_hbm.at[ti_v.at[k]], t_v.at[pl.ds(k * IC, IC)], sem))
        copies.append(pltpu.async_copy(
            rel_hbm.at[ri_v.at[k]], r_v.at[pl.ds(k * IC, IC)], sem))
    for c in copies:
        c.wait()

    mask_hi = jnp.int32(-65536)

    def row_body(rr, _):
        acc = None
        for k in range(2):
            sl = pl.ds(k * 16, 16)
            hw = h_v[rr, sl]
            rw = r_v[rr, sl]
            tw = t_v[rr, sl]
            for part in range(2):
                if part == 0:
                    hf = lax.bitcast_convert_type(lax.shift_left(hw, 16), jnp.float32)
                    rf = lax.bitcast_convert_type(lax.shift_left(rw, 16), jnp.float32)
                    tf = lax.bitcast_convert_type(lax.shift_left(tw, 16), jnp.float32)
                else:
                    hf = lax.bitcast_convert_type(hw & mask_hi, jnp.float32)
                    rf = lax.bitcast_convert_type(rw & mask_hi, jnp.float32)
                    tf = lax.bitcast_convert_type(tw & mask_hi, jnp.float32)
                d = jnp.abs(hf + rf - tf)
                acc = d if acc is None else acc + d
        csum_v[pl.ds(rr * 16, 16)] = acc
        return 0

    lax.fori_loop(0, BW, row_body, 0)

    pltpu.sync_copy(csum_v, slab.at[sid])
    tcopies = []
    for k in range(NT):
        tcopies.append(pltpu.async_copy(
            slab.at[sid].at[perm_v.at[k]],
            csum_v.at[pl.ds(k * IC, IC)], sem))
    for c in tcopies:
        c.wait()

    def grp_body(g, _):
        acc = None
        for c in range(16):
            v = csum_v[pl.ds(c * BW + g * 16, 16)]
            acc = v if acc is None else acc + v
        out_v[pl.ds(g * 16, 16)] = -acc
        return 0

    lax.fori_loop(0, BW // 16, grp_body, 0)

    pltpu.sync_copy(out_v, out_hbm.at[pl.ds(wid * BW, BW)])


@jax.jit
def _transe(h_idx, t_idx, r_idx, entity_table, relation_table):
    ent500 = _tc_transpose(entity_table.T, V)
    rel500 = _tc_transpose(relation_table.T, RN)
    ent2 = ent500.reshape(-1, 32)
    rel2 = rel500.reshape(-1, 32)

    def _qidx(idx):
        i = idx.astype(jnp.int32)
        q = CI // 4
        return ((i // CI) * CI + 4 * (i % q) + (i % CI) // q).reshape(-1, IC)

    k = jnp.arange(CS, dtype=jnp.int32)
    perm = ((k % BW) * 16 + k // BW).reshape(NT, IC)

    kfn = pl.kernel(
        _sc_body,
        out_type=jax.ShapeDtypeStruct((B,), jnp.float32),
        mesh=plsc.VectorSubcoreMesh(
            core_axis_name="c", subcore_axis_name="s",
            num_cores=NC, num_subcores=NS),
        compiler_params=pltpu.CompilerParams(use_tc_tiling_on_sc=False),
        scratch_types=[
            pltpu.VMEM((NCH, IC), jnp.int32),
            pltpu.VMEM((NCH, IC), jnp.int32),
            pltpu.VMEM((NCH, IC), jnp.int32),
            pltpu.VMEM((BW, 32), jnp.int32),
            pltpu.VMEM((BW, 32), jnp.int32),
            pltpu.VMEM((BW, 32), jnp.int32),
            pltpu.VMEM((CS,), jnp.float32),
            pltpu.VMEM((NT, IC), jnp.int32),
            pltpu.VMEM((BW,), jnp.float32),
            pltpu.VMEM_SHARED((NS, CS), jnp.float32),
            pltpu.SemaphoreType.DMA,
        ],
    )
    return kfn(_qidx(h_idx), _qidx(t_idx), _qidx(r_idx),
               ent2, rel2, perm)


def kernel(h_idx, t_idx, r_idx, entity_table, relation_table):
    return _transe(h_idx, t_idx, r_idx, entity_table, relation_table)
